# baseline (speedup 1.0000x reference)
import jax
import jax.numpy as jnp
from jax import lax
from jax.experimental import pallas as pl
from jax.experimental.pallas import tpu as pltpu
from jax.experimental.pallas import tpu_sc as plsc

N = 100000
E = 1600000
F_IN = 10
HID = 64
NC, NS, L = 2, 16, 16

NACC = 102400
SLICE = NACC // NS
PK = NACC * L // 128

K = 4
GRP = K * 128
EPAD = 1638400
G32 = 100
RW = G32 * K + 2 * K
ROWS = 32 * RW


def _zero_acc(acc, sid, zeros_hbm):
    pltpu.sync_copy(zeros_hbm.at[pl.ds(sid * SLICE, SLICE)],
                    acc.at[pl.ds(sid * SLICE, SLICE)])


def _copy_out(acc, out, cid, sid):
    pltpu.sync_copy(
        acc.at[pl.ds(sid * SLICE, SLICE)],
        out.at[cid, pl.ds(sid * SLICE, SLICE)],
    )


def _deg_body(dst_hbm, zeros_hbm, ones_hbm, out_hbm, didx, ones, acc):
    cid = lax.axis_index("c")
    sid = lax.axis_index("s")
    pltpu.sync_copy(ones_hbm, ones)
    _zero_acc(acc, sid, zeros_hbm)
    plsc.subcore_barrier()

    wid = cid * NS + sid

    def group(g, _):
        rb = wid * RW + g * K
        pltpu.sync_copy(dst_hbm.at[pl.ds(rb, K)], didx)
        for j in range(K):
            pltpu.sync_copy(ones, acc.at[didx.at[j]], add=True)
        return 0

    lax.fori_loop(0, G32, group, 0)
    plsc.subcore_barrier()
    _copy_out(acc, out_hbm, cid, sid)


def _agg_sweep(table_hbm, src_hbm, dst_hbm, zeros_hbm, acc, sidx2, didx2,
               rows2, gsem, wid):
    base = wid * RW
    pltpu.sync_copy(zeros_hbm.at[pl.ds(0, 2 * GRP)], rows2)
    pltpu.sync_copy(dst_hbm.at[pl.ds(base + G32 * K, 2 * K)], didx2)

    def step(g, _):
        p = lax.rem(g, 2)
        o = p * K
        pltpu.sync_copy(src_hbm.at[pl.ds(base + g * K, K)],
                        sidx2.at[pl.ds(o, K)])
        handles = [
            pltpu.async_copy(table_hbm.at[sidx2.at[o + j]],
                             rows2.at[pl.ds((o + j) * 128, 128)], gsem)
            for j in range(K)
        ]
        oo = K - o
        for j in range(K):
            pltpu.sync_copy(rows2.at[pl.ds((oo + j) * 128, 128)],
                            acc.at[didx2.at[oo + j]], add=True)
        for h in handles:
            h.wait()
        pltpu.sync_copy(dst_hbm.at[pl.ds(base + g * K, K)],
                        didx2.at[pl.ds(o, K)])
        return 0

    lax.fori_loop(0, G32 + 1, step, 0)


def _agg_body(table_hbm, src_hbm, dst_hbm, zeros_hbm, out_hbm, sidx2, didx2,
              rows2, acc, gsem):
    cid = lax.axis_index("c")
    sid = lax.axis_index("s")
    _zero_acc(acc, sid, zeros_hbm)
    plsc.subcore_barrier()

    wid = cid * NS + sid
    _agg_sweep(table_hbm, src_hbm, dst_hbm, zeros_hbm, acc, sidx2, didx2,
               rows2, gsem, wid)
    plsc.subcore_barrier()
    _copy_out(acc, out_hbm, cid, sid)


def _agg4_body(t0, t1, t2, t3, src_hbm, dst_hbm, zeros_hbm, out_hbm, sidx2,
               didx2, rows2, acc, gsem):
    cid = lax.axis_index("c")
    sid = lax.axis_index("s")
    wid = cid * NS + sid

    for c, table_hbm in enumerate((t0, t1, t2, t3)):
        _zero_acc(acc, sid, zeros_hbm)
        plsc.subcore_barrier()
        _agg_sweep(table_hbm, src_hbm, dst_hbm, zeros_hbm, acc, sidx2, didx2,
                   rows2, gsem, wid)
        plsc.subcore_barrier()
        pltpu.sync_copy(
            acc.at[pl.ds(sid * SLICE, SLICE)],
            out_hbm.at[c, cid, pl.ds(sid * SLICE, SLICE)],
        )
        plsc.subcore_barrier()


def _sc_mesh():
    return plsc.VectorSubcoreMesh(core_axis_name="c", subcore_axis_name="s")


_SC_PARAMS = pltpu.CompilerParams(use_tc_tiling_on_sc=False)


def _sc_deg(dst2d, zeros_hbm, ones_hbm):
    fn = pl.kernel(
        _deg_body,
        out_type=jax.ShapeDtypeStruct((NC, NACC, L), jnp.float32),
        mesh=_sc_mesh(),
        compiler_params=_SC_PARAMS,
        scratch_types=[
            pltpu.VMEM((K, 128), jnp.int32),
            pltpu.VMEM((128, L), jnp.float32),
            pltpu.VMEM_SHARED((NACC, L), jnp.float32),
        ],
    )
    return fn(dst2d, zeros_hbm, ones_hbm)


def _sc_agg(table, src2d, dst2d, zeros_hbm):
    fn = pl.kernel(
        _agg_body,
        out_type=jax.ShapeDtypeStruct((NC, NACC, L), jnp.float32),
        mesh=_sc_mesh(),
        compiler_params=_SC_PARAMS,
        scratch_types=[
            pltpu.VMEM((2 * K, 128), jnp.int32),
            pltpu.VMEM((2 * K, 128), jnp.int32),
            pltpu.VMEM((2 * GRP, L), jnp.float32),
            pltpu.VMEM_SHARED((NACC, L), jnp.float32),
            pltpu.SemaphoreType.DMA,
        ],
    )
    return fn(table, src2d, dst2d, zeros_hbm)


def _sc_agg4(t0, t1, t2, t3, src2d, dst2d, zeros_hbm):
    fn = pl.kernel(
        _agg4_body,
        out_type=jax.ShapeDtypeStruct((4, NC, NACC, L), jnp.float32),
        mesh=_sc_mesh(),
        compiler_params=_SC_PARAMS,
        scratch_types=[
            pltpu.VMEM((2 * K, 128), jnp.int32),
            pltpu.VMEM((2 * K, 128), jnp.int32),
            pltpu.VMEM((2 * GRP, L), jnp.float32),
            pltpu.VMEM_SHARED((NACC, L), jnp.float32),
            pltpu.SemaphoreType.DMA,
        ],
    )
    return fn(t0, t1, t2, t3, src2d, dst2d, zeros_hbm)


PBLK = 256
GRID = PK // PBLK


def _pspec():
    return pl.BlockSpec((PBLK, 128), lambda i: (i, 0))


def _full(shape):
    return pl.BlockSpec(shape, lambda i: tuple(0 for _ in shape))


def _tc_a_body(dp, xpk, bd_wtop, xs_ref, dinv_ref, xtop_ref):
    dinv = lax.rsqrt(dp[0] + dp[1] + 1.0)
    dinv_ref[...] = dinv
    xs_ref[...] = xpk[...] * dinv
    xtop_ref[...] = jnp.dot(xpk[...], bd_wtop[...],
                            preferred_element_type=jnp.float32)


def _tc_a(degp_p, xpk, bd_wtop):
    return pl.pallas_call(
        _tc_a_body,
        grid=(GRID,),
        in_specs=[
            pl.BlockSpec((NC, PBLK, 128), lambda i: (0, i, 0)),
            _pspec(),
            _full((128, 8)),
        ],
        out_specs=[_pspec(), _pspec(), pl.BlockSpec((PBLK, 8), lambda i: (i, 0))],
        out_shape=[
            jax.ShapeDtypeStruct((PK, 128), jnp.float32),
            jax.ShapeDtypeStruct((PK, 128), jnp.float32),
            jax.ShapeDtypeStruct((PK, 8), jnp.float32),
        ],
    )(degp_p, xpk, bd_wtop)


def _tc_b_body(a1p, xs, dinv, bd_w1, bd_m, b1t, sel0, sel1, sel2, sel3,
               h0, h1, h2, h3):
    u = (a1p[0] + a1p[1] + xs[...]) * dinv[...]
    h = jnp.dot(u, bd_w1[...], preferred_element_type=jnp.float32) + b1t[...]
    h = jnp.maximum(h, 0.0)
    dinv64 = jnp.dot(dinv[...], bd_m[...], preferred_element_type=jnp.float32)
    hs = h * dinv64
    for ref, sel in ((h0, sel0), (h1, sel1), (h2, sel2), (h3, sel3)):
        ref[...] = jnp.dot(hs, sel[...], preferred_element_type=jnp.float32)


def _tc_b(a1p_p, xs, dinv, bd_w1, bd_m, b1t, sels):
    return pl.pallas_call(
        _tc_b_body,
        grid=(GRID,),
        in_specs=[
            pl.BlockSpec((NC, PBLK, 128), lambda i: (0, i, 0)),
            _pspec(), _pspec(),
            _full((128, 8 * HID)), _full((128, 8 * HID)), _full((1, 8 * HID)),
            _full((8 * HID, 128)), _full((8 * HID, 128)),
            _full((8 * HID, 128)), _full((8 * HID, 128)),
        ],
        out_specs=[_pspec()] * 4,
        out_shape=[jax.ShapeDtypeStruct((PK, 128), jnp.float32)] * 4,
    )(a1p_p, xs, dinv, bd_w1, bd_m, b1t, *sels)


def _tc_c_body(a2p, h0, h1, h2, h3, dinv, xtop, w0, w1, w2, w3, b2t, bd_wbot,
               bfc, out_ref):
    hs = (h0, h1, h2, h3)
    ws = (w0, w1, w2, w3)
    acc = b2t[...]
    for c in range(4):
        a2c = (a2p[c, 0] + a2p[c, 1] + hs[c][...]) * dinv[...]
        acc = acc + jnp.dot(a2c, ws[c][...],
                            preferred_element_type=jnp.float32)
    x2 = jnp.maximum(acc, 0.0)
    out_ref[...] = (xtop[...]
                    + jnp.dot(x2, bd_wbot[...],
                              preferred_element_type=jnp.float32)
                    + bfc[...])


def _tc_c(a2p_p, h1s_p, dinv, xtop, bd_w2, b2t, bd_wbot, bfc):
    return pl.pallas_call(
        _tc_c_body,
        grid=(GRID,),
        in_specs=[
            pl.BlockSpec((4, NC, PBLK, 128), lambda i: (0, 0, i, 0)),
            _pspec(), _pspec(), _pspec(), _pspec(),
            _pspec(),
            pl.BlockSpec((PBLK, 8), lambda i: (i, 0)),
            _full((128, 8 * 2 * HID)), _full((128, 8 * 2 * HID)),
            _full((128, 8 * 2 * HID)), _full((128, 8 * 2 * HID)),
            _full((1, 8 * 2 * HID)),
            _full((8 * 2 * HID, 8)),
            _full((1, 1)),
        ],
        out_specs=pl.BlockSpec((PBLK, 8), lambda i: (i, 0)),
        out_shape=jax.ShapeDtypeStruct((PK, 8), jnp.float32),
    )(a2p_p, *h1s_p, dinv, xtop, *bd_w2, b2t, bd_wbot, bfc)


def kernel(x, edge_index, W1, b1, W2, b2, Wfc, bfc):
    f32 = jnp.float32
    src = edge_index[0].astype(jnp.int32)
    dst = edge_index[1].astype(jnp.int32)
    npad = EPAD - E
    pad_i = jnp.arange(npad, dtype=jnp.int32)
    pad_src = (pad_i * 641) % N
    pad_dst = N + (pad_i % (NACC - N))
    dum_i = jnp.arange(32 * 2 * K * 128, dtype=jnp.int32)
    dum_src = (dum_i * 389) % N
    dum_dst = N + (dum_i % (NACC - N))
    src2d = jnp.concatenate([
        jnp.concatenate([src, pad_src]).reshape(32, G32 * K, 128),
        dum_src.reshape(32, 2 * K, 128),
    ], axis=1).reshape(ROWS, 128)
    dst2d = jnp.concatenate([
        jnp.concatenate([dst, pad_dst]).reshape(32, G32 * K, 128),
        dum_dst.reshape(32, 2 * K, 128),
    ], axis=1).reshape(ROWS, 128)

    zeros_hbm = jnp.zeros((NACC, L), f32)
    ones_hbm = jnp.ones((128, L), f32)

    xpk = jnp.pad(x, ((0, NACC - N), (0, L - F_IN))).reshape(PK, 128)

    eye8 = jnp.eye(8, dtype=f32)
    wtop16 = jnp.pad(Wfc[:F_IN], ((0, L - F_IN), (0, 0)))
    bd_wtop = jnp.kron(eye8, wtop16)
    w1p = jnp.pad(W1, ((0, L - F_IN), (0, 0)))
    bd_w1 = jnp.kron(eye8, w1p)
    m16 = jnp.zeros((L, HID), f32).at[0, :].set(1.0)
    bd_m = jnp.kron(eye8, m16)
    b1t = jnp.tile(b1, 8).reshape(1, 8 * HID)
    sels = []
    for c in range(4):
        ec = jnp.zeros((HID, L), f32).at[c * L + jnp.arange(L),
                                         jnp.arange(L)].set(1.0)
        sels.append(jnp.kron(eye8, ec))
    bd_w2 = [jnp.kron(eye8, W2[c * L:(c + 1) * L]) for c in range(4)]
    b2t = jnp.tile(b2, 8).reshape(1, 8 * 2 * HID)
    bd_wbot = jnp.kron(eye8, Wfc[F_IN:])
    bfc2 = bfc.reshape(1, 1)

    degp = _sc_deg(dst2d, zeros_hbm, ones_hbm)
    degp_p = degp.reshape(NC, PK, 128)

    xs_p, dinv_p, xtop_p = _tc_a(degp_p, xpk, bd_wtop)

    a1p = _sc_agg(xs_p.reshape(NACC, L), src2d, dst2d, zeros_hbm)

    h1s_p = _tc_b(a1p.reshape(NC, PK, 128), xs_p, dinv_p, bd_w1, bd_m, b1t,
                  sels)

    a2p = _sc_agg4(h1s_p[0].reshape(NACC, L), h1s_p[1].reshape(NACC, L),
                   h1s_p[2].reshape(NACC, L), h1s_p[3].reshape(NACC, L),
                   src2d, dst2d, zeros_hbm)

    out = _tc_c(a2p.reshape(4, NC, PK, 128), h1s_p, dinv_p, xtop_p, bd_w2,
                b2t, bd_wbot, bfc2)
    return out.reshape(NACC)[:N]


# speedup vs baseline: 1.1948x; 1.1948x over previous
import jax
import jax.numpy as jnp
from jax import lax
from jax.experimental import pallas as pl
from jax.experimental.pallas import tpu as pltpu
from jax.experimental.pallas import tpu_sc as plsc

N = 100000
E = 1600000
F_IN = 10
HID = 64
NC, NS, L = 2, 16, 16

NACC = 102400
SLICE = NACC // NS
PK = NACC * L // 128

K = 6
GRP = K * 128
EPAD = 1622016
G32 = 66
RW = G32 * K + 2 * K
ROWS = 32 * RW


def _zero_acc(acc, sid, zeros_hbm):
    pltpu.sync_copy(zeros_hbm.at[pl.ds(sid * SLICE, SLICE)],
                    acc.at[pl.ds(sid * SLICE, SLICE)])


def _copy_out(acc, out, cid, sid):
    pltpu.sync_copy(
        acc.at[pl.ds(sid * SLICE, SLICE)],
        out.at[cid, pl.ds(sid * SLICE, SLICE)],
    )


def _deg_body(dst_hbm, zeros_hbm, ones_hbm, out_hbm, didx, ones, acc):
    cid = lax.axis_index("c")
    sid = lax.axis_index("s")
    pltpu.sync_copy(ones_hbm, ones)
    _zero_acc(acc, sid, zeros_hbm)
    plsc.subcore_barrier()

    wid = cid * NS + sid

    def group(g, _):
        rb = wid * RW + g * K
        pltpu.sync_copy(dst_hbm.at[pl.ds(rb, K)], didx)
        for j in range(K):
            pltpu.sync_copy(ones, acc.at[didx.at[j]], add=True)
        return 0

    lax.fori_loop(0, G32, group, 0)
    plsc.subcore_barrier()
    _copy_out(acc, out_hbm, cid, sid)


def _agg_sweep(table_hbm, src_hbm, dst_hbm, zeros_hbm, acc, sidx2, didx2,
               rows2, gsem, wid):
    base = wid * RW
    pltpu.sync_copy(zeros_hbm.at[pl.ds(0, 2 * GRP)], rows2)
    pltpu.sync_copy(dst_hbm.at[pl.ds(base + G32 * K, 2 * K)], didx2)

    def step(g, _):
        p = lax.rem(g, 2)
        o = p * K
        pltpu.sync_copy(src_hbm.at[pl.ds(base + g * K, K)],
                        sidx2.at[pl.ds(o, K)])
        handles = [
            pltpu.async_copy(table_hbm.at[sidx2.at[o + j]],
                             rows2.at[pl.ds((o + j) * 128, 128)], gsem)
            for j in range(K)
        ]
        oo = K - o
        for j in range(K):
            pltpu.sync_copy(rows2.at[pl.ds((oo + j) * 128, 128)],
                            acc.at[didx2.at[oo + j]], add=True)
        for h in handles:
            h.wait()
        pltpu.sync_copy(dst_hbm.at[pl.ds(base + g * K, K)],
                        didx2.at[pl.ds(o, K)])
        return 0

    lax.fori_loop(0, G32 + 1, step, 0)


def _agg_body(table_hbm, src_hbm, dst_hbm, zeros_hbm, out_hbm, sidx2, didx2,
              rows2, acc, gsem):
    cid = lax.axis_index("c")
    sid = lax.axis_index("s")
    _zero_acc(acc, sid, zeros_hbm)
    plsc.subcore_barrier()

    wid = cid * NS + sid
    _agg_sweep(table_hbm, src_hbm, dst_hbm, zeros_hbm, acc, sidx2, didx2,
               rows2, gsem, wid)
    plsc.subcore_barrier()
    _copy_out(acc, out_hbm, cid, sid)


def _agg4_body(t0, t1, t2, t3, src_hbm, dst_hbm, zeros_hbm, out_hbm, sidx2,
               didx2, rows2, acc, gsem):
    cid = lax.axis_index("c")
    sid = lax.axis_index("s")
    wid = cid * NS + sid

    for c, table_hbm in enumerate((t0, t1, t2, t3)):
        _zero_acc(acc, sid, zeros_hbm)
        plsc.subcore_barrier()
        _agg_sweep(table_hbm, src_hbm, dst_hbm, zeros_hbm, acc, sidx2, didx2,
                   rows2, gsem, wid)
        plsc.subcore_barrier()
        pltpu.sync_copy(
            acc.at[pl.ds(sid * SLICE, SLICE)],
            out_hbm.at[c, cid, pl.ds(sid * SLICE, SLICE)],
        )
        plsc.subcore_barrier()


def _sc_mesh():
    return plsc.VectorSubcoreMesh(core_axis_name="c", subcore_axis_name="s")


_SC_PARAMS = pltpu.CompilerParams(use_tc_tiling_on_sc=False)


def _sc_deg(dst2d, zeros_hbm, ones_hbm):
    fn = pl.kernel(
        _deg_body,
        out_type=jax.ShapeDtypeStruct((NC, NACC, L), jnp.float32),
        mesh=_sc_mesh(),
        compiler_params=_SC_PARAMS,
        scratch_types=[
            pltpu.VMEM((K, 128), jnp.int32),
            pltpu.VMEM((128, L), jnp.float32),
            pltpu.VMEM_SHARED((NACC, L), jnp.float32),
        ],
    )
    return fn(dst2d, zeros_hbm, ones_hbm)


def _sc_agg(table, src2d, dst2d, zeros_hbm):
    fn = pl.kernel(
        _agg_body,
        out_type=jax.ShapeDtypeStruct((NC, NACC, L), jnp.float32),
        mesh=_sc_mesh(),
        compiler_params=_SC_PARAMS,
        scratch_types=[
            pltpu.VMEM((2 * K, 128), jnp.int32),
            pltpu.VMEM((2 * K, 128), jnp.int32),
            pltpu.VMEM((2 * GRP, L), jnp.float32),
            pltpu.VMEM_SHARED((NACC, L), jnp.float32),
            pltpu.SemaphoreType.DMA,
        ],
    )
    return fn(table, src2d, dst2d, zeros_hbm)


def _sc_agg4(t0, t1, t2, t3, src2d, dst2d, zeros_hbm):
    fn = pl.kernel(
        _agg4_body,
        out_type=jax.ShapeDtypeStruct((4, NC, NACC, L), jnp.float32),
        mesh=_sc_mesh(),
        compiler_params=_SC_PARAMS,
        scratch_types=[
            pltpu.VMEM((2 * K, 128), jnp.int32),
            pltpu.VMEM((2 * K, 128), jnp.int32),
            pltpu.VMEM((2 * GRP, L), jnp.float32),
            pltpu.VMEM_SHARED((NACC, L), jnp.float32),
            pltpu.SemaphoreType.DMA,
        ],
    )
    return fn(t0, t1, t2, t3, src2d, dst2d, zeros_hbm)


PBLK = 256
GRID = PK // PBLK


def _pspec():
    return pl.BlockSpec((PBLK, 128), lambda i: (i, 0))


def _full(shape):
    return pl.BlockSpec(shape, lambda i: tuple(0 for _ in shape))


def _tc_a_body(dp, xpk, bd_wtop, xs_ref, dinv_ref, xtop_ref):
    dinv = lax.rsqrt(dp[0] + dp[1] + 1.0)
    dinv_ref[...] = dinv
    xs_ref[...] = xpk[...] * dinv
    xtop_ref[...] = jnp.dot(xpk[...], bd_wtop[...],
                            preferred_element_type=jnp.float32)


def _tc_a(degp_p, xpk, bd_wtop):
    return pl.pallas_call(
        _tc_a_body,
        grid=(GRID,),
        in_specs=[
            pl.BlockSpec((NC, PBLK, 128), lambda i: (0, i, 0)),
            _pspec(),
            _full((128, 8)),
        ],
        out_specs=[_pspec(), _pspec(), pl.BlockSpec((PBLK, 8), lambda i: (i, 0))],
        out_shape=[
            jax.ShapeDtypeStruct((PK, 128), jnp.float32),
            jax.ShapeDtypeStruct((PK, 128), jnp.float32),
            jax.ShapeDtypeStruct((PK, 8), jnp.float32),
        ],
    )(degp_p, xpk, bd_wtop)


def _tc_b_body(a1p, xs, dinv, bd_w1, bd_m, b1t, sel0, sel1, sel2, sel3,
               h0, h1, h2, h3):
    u = (a1p[0] + a1p[1] + xs[...]) * dinv[...]
    h = jnp.dot(u, bd_w1[...], preferred_element_type=jnp.float32) + b1t[...]
    h = jnp.maximum(h, 0.0)
    dinv64 = jnp.dot(dinv[...], bd_m[...], preferred_element_type=jnp.float32)
    hs = h * dinv64
    for ref, sel in ((h0, sel0), (h1, sel1), (h2, sel2), (h3, sel3)):
        ref[...] = jnp.dot(hs, sel[...], preferred_element_type=jnp.float32)


def _tc_b(a1p_p, xs, dinv, bd_w1, bd_m, b1t, sels):
    return pl.pallas_call(
        _tc_b_body,
        grid=(GRID,),
        in_specs=[
            pl.BlockSpec((NC, PBLK, 128), lambda i: (0, i, 0)),
            _pspec(), _pspec(),
            _full((128, 8 * HID)), _full((128, 8 * HID)), _full((1, 8 * HID)),
            _full((8 * HID, 128)), _full((8 * HID, 128)),
            _full((8 * HID, 128)), _full((8 * HID, 128)),
        ],
        out_specs=[_pspec()] * 4,
        out_shape=[jax.ShapeDtypeStruct((PK, 128), jnp.float32)] * 4,
    )(a1p_p, xs, dinv, bd_w1, bd_m, b1t, *sels)


def _tc_c_body(a2p, h0, h1, h2, h3, dinv, xtop, w0, w1, w2, w3, b2t, bd_wbot,
               bfc, out_ref):
    hs = (h0, h1, h2, h3)
    ws = (w0, w1, w2, w3)
    acc = b2t[...]
    for c in range(4):
        a2c = (a2p[c, 0] + a2p[c, 1] + hs[c][...]) * dinv[...]
        acc = acc + jnp.dot(a2c, ws[c][...],
                            preferred_element_type=jnp.float32)
    x2 = jnp.maximum(acc, 0.0)
    out_ref[...] = (xtop[...]
                    + jnp.dot(x2, bd_wbot[...],
                              preferred_element_type=jnp.float32)
                    + bfc[...])


def _tc_c(a2p_p, h1s_p, dinv, xtop, bd_w2, b2t, bd_wbot, bfc):
    return pl.pallas_call(
        _tc_c_body,
        grid=(GRID,),
        in_specs=[
            pl.BlockSpec((4, NC, PBLK, 128), lambda i: (0, 0, i, 0)),
            _pspec(), _pspec(), _pspec(), _pspec(),
            _pspec(),
            pl.BlockSpec((PBLK, 8), lambda i: (i, 0)),
            _full((128, 8 * 2 * HID)), _full((128, 8 * 2 * HID)),
            _full((128, 8 * 2 * HID)), _full((128, 8 * 2 * HID)),
            _full((1, 8 * 2 * HID)),
            _full((8 * 2 * HID, 8)),
            _full((1, 1)),
        ],
        out_specs=pl.BlockSpec((PBLK, 8), lambda i: (i, 0)),
        out_shape=jax.ShapeDtypeStruct((PK, 8), jnp.float32),
    )(a2p_p, *h1s_p, dinv, xtop, *bd_w2, b2t, bd_wbot, bfc)


def kernel(x, edge_index, W1, b1, W2, b2, Wfc, bfc):
    f32 = jnp.float32
    src = edge_index[0].astype(jnp.int32)
    dst = edge_index[1].astype(jnp.int32)
    npad = EPAD - E
    pad_i = jnp.arange(npad, dtype=jnp.int32)
    pad_src = (pad_i * 641) % N
    pad_dst = N + (pad_i % (NACC - N))
    dum_i = jnp.arange(32 * 2 * K * 128, dtype=jnp.int32)
    dum_src = (dum_i * 389) % N
    dum_dst = N + (dum_i % (NACC - N))
    src2d = jnp.concatenate([
        jnp.concatenate([src, pad_src]).reshape(32, G32 * K, 128),
        dum_src.reshape(32, 2 * K, 128),
    ], axis=1).reshape(ROWS, 128)
    dst2d = jnp.concatenate([
        jnp.concatenate([dst, pad_dst]).reshape(32, G32 * K, 128),
        dum_dst.reshape(32, 2 * K, 128),
    ], axis=1).reshape(ROWS, 128)

    zeros_hbm = jnp.zeros((NACC, L), f32)
    ones_hbm = jnp.ones((128, L), f32)

    xpk = jnp.pad(x, ((0, NACC - N), (0, L - F_IN))).reshape(PK, 128)

    eye8 = jnp.eye(8, dtype=f32)
    wtop16 = jnp.pad(Wfc[:F_IN], ((0, L - F_IN), (0, 0)))
    bd_wtop = jnp.kron(eye8, wtop16)
    w1p = jnp.pad(W1, ((0, L - F_IN), (0, 0)))
    bd_w1 = jnp.kron(eye8, w1p)
    m16 = jnp.zeros((L, HID), f32).at[0, :].set(1.0)
    bd_m = jnp.kron(eye8, m16)
    b1t = jnp.tile(b1, 8).reshape(1, 8 * HID)
    sels = []
    for c in range(4):
        ec = jnp.zeros((HID, L), f32).at[c * L + jnp.arange(L),
                                         jnp.arange(L)].set(1.0)
        sels.append(jnp.kron(eye8, ec))
    bd_w2 = [jnp.kron(eye8, W2[c * L:(c + 1) * L]) for c in range(4)]
    b2t = jnp.tile(b2, 8).reshape(1, 8 * 2 * HID)
    bd_wbot = jnp.kron(eye8, Wfc[F_IN:])
    bfc2 = bfc.reshape(1, 1)

    degp = _sc_deg(dst2d, zeros_hbm, ones_hbm)
    degp_p = degp.reshape(NC, PK, 128)

    xs_p, dinv_p, xtop_p = _tc_a(degp_p, xpk, bd_wtop)

    a1p = _sc_agg(xs_p.reshape(NACC, L), src2d, dst2d, zeros_hbm)

    h1s_p = _tc_b(a1p.reshape(NC, PK, 128), xs_p, dinv_p, bd_w1, bd_m, b1t,
                  sels)

    a2p = _sc_agg4(h1s_p[0].reshape(NACC, L), h1s_p[1].reshape(NACC, L),
                   h1s_p[2].reshape(NACC, L), h1s_p[3].reshape(NACC, L),
                   src2d, dst2d, zeros_hbm)

    out = _tc_c(a2p.reshape(4, NC, PK, 128), h1s_p, dinv_p, xtop_p, bd_w2,
                b2t, bd_wbot, bfc2)
    return out.reshape(NACC)[:N]


# speedup vs baseline: 1.5149x; 1.2679x over previous
import jax
import jax.numpy as jnp
from jax import lax
from jax.experimental import pallas as pl
from jax.experimental.pallas import tpu as pltpu
from jax.experimental.pallas import tpu_sc as plsc

N = 100000
E = 1600000
F_IN = 10
HID = 64
NC, NS, L = 2, 16, 16

NACC = 102400
SLICE = NACC // NS
PK = NACC * L // 128

K = 6
GRP = K * 128
EPAD = 1622016
G32 = 66
RW = (G32 + 2) * 2 * K
ROWS = 32 * RW


def _zero_acc(acc, sid, zeros_hbm):
    pltpu.sync_copy(zeros_hbm.at[pl.ds(sid * SLICE, SLICE)],
                    acc.at[pl.ds(sid * SLICE, SLICE)])


def _copy_out(acc, out, cid, sid):
    pltpu.sync_copy(
        acc.at[pl.ds(sid * SLICE, SLICE)],
        out.at[cid, pl.ds(sid * SLICE, SLICE)],
    )


def _deg_body(sd_hbm, zeros_hbm, ones_hbm, out_hbm, didx, ones, acc):
    cid = lax.axis_index("c")
    sid = lax.axis_index("s")
    pltpu.sync_copy(ones_hbm, ones)
    _zero_acc(acc, sid, zeros_hbm)
    plsc.subcore_barrier()

    wid = cid * NS + sid

    def group(g, _):
        rb = wid * RW + g * 2 * K + K
        pltpu.sync_copy(sd_hbm.at[pl.ds(rb, K)], didx)
        for j in range(K):
            pltpu.sync_copy(ones, acc.at[didx.at[j]], add=True)
        return 0

    lax.fori_loop(0, G32, group, 0)
    plsc.subcore_barrier()
    _copy_out(acc, out_hbm, cid, sid)


def _agg_sweep(table_hbm, sd_hbm, zeros_hbm, acc, idx4, rows2, gsem, isem,
               wid):
    base = wid * RW
    pltpu.sync_copy(zeros_hbm.at[pl.ds(0, 2 * GRP)], rows2)
    pltpu.sync_copy(sd_hbm.at[pl.ds(base + G32 * 2 * K, 2 * K)],
                    idx4.at[pl.ds(2 * K, 2 * K)])
    pltpu.async_copy(sd_hbm.at[pl.ds(base, 2 * K)],
                     idx4.at[pl.ds(0, 2 * K)], isem)

    def step(g, _):
        p = lax.rem(g, 2)
        o = p * 2 * K
        ro = p * K
        pltpu.make_async_copy(sd_hbm.at[pl.ds(0, 2 * K)],
                              idx4.at[pl.ds(o, 2 * K)], isem).wait()
        handles = [
            pltpu.async_copy(table_hbm.at[idx4.at[o + j]],
                             rows2.at[pl.ds((ro + j) * 128, 128)], gsem)
            for j in range(K)
        ]
        oo = 2 * K - o
        roo = K - ro
        for j in range(K):
            pltpu.sync_copy(rows2.at[pl.ds((roo + j) * 128, 128)],
                            acc.at[idx4.at[oo + K + j]], add=True)
        pltpu.async_copy(sd_hbm.at[pl.ds(base + (g + 1) * 2 * K, 2 * K)],
                         idx4.at[pl.ds(oo, 2 * K)], isem)
        for h in handles:
            h.wait()
        return 0

    lax.fori_loop(0, G32 + 1, step, 0)
    pltpu.make_async_copy(sd_hbm.at[pl.ds(0, 2 * K)],
                          idx4.at[pl.ds(0, 2 * K)], isem).wait()


def _agg_body(table_hbm, sd_hbm, zeros_hbm, out_hbm, idx4, rows2, acc, gsem,
              isem):
    cid = lax.axis_index("c")
    sid = lax.axis_index("s")
    _zero_acc(acc, sid, zeros_hbm)
    plsc.subcore_barrier()

    wid = cid * NS + sid
    _agg_sweep(table_hbm, sd_hbm, zeros_hbm, acc, idx4, rows2, gsem, isem,
               wid)
    plsc.subcore_barrier()
    _copy_out(acc, out_hbm, cid, sid)


def _agg4_body(t0, t1, t2, t3, sd_hbm, zeros_hbm, out_hbm, idx4, rows2, acc,
               gsem, isem):
    cid = lax.axis_index("c")
    sid = lax.axis_index("s")
    wid = cid * NS + sid

    for c, table_hbm in enumerate((t0, t1, t2, t3)):
        _zero_acc(acc, sid, zeros_hbm)
        plsc.subcore_barrier()
        _agg_sweep(table_hbm, sd_hbm, zeros_hbm, acc, idx4, rows2, gsem,
                   isem, wid)
        plsc.subcore_barrier()
        pltpu.sync_copy(
            acc.at[pl.ds(sid * SLICE, SLICE)],
            out_hbm.at[c, cid, pl.ds(sid * SLICE, SLICE)],
        )
        plsc.subcore_barrier()


def _sc_mesh():
    return plsc.VectorSubcoreMesh(core_axis_name="c", subcore_axis_name="s")


_SC_PARAMS = pltpu.CompilerParams(use_tc_tiling_on_sc=False)


def _sc_deg(dst2d, zeros_hbm, ones_hbm):
    fn = pl.kernel(
        _deg_body,
        out_type=jax.ShapeDtypeStruct((NC, NACC, L), jnp.float32),
        mesh=_sc_mesh(),
        compiler_params=_SC_PARAMS,
        scratch_types=[
            pltpu.VMEM((K, 128), jnp.int32),
            pltpu.VMEM((128, L), jnp.float32),
            pltpu.VMEM_SHARED((NACC, L), jnp.float32),
        ],
    )
    return fn(dst2d, zeros_hbm, ones_hbm)


def _sc_agg(table, sd2d, zeros_hbm):
    fn = pl.kernel(
        _agg_body,
        out_type=jax.ShapeDtypeStruct((NC, NACC, L), jnp.float32),
        mesh=_sc_mesh(),
        compiler_params=_SC_PARAMS,
        scratch_types=[
            pltpu.VMEM((4 * K, 128), jnp.int32),
            pltpu.VMEM((2 * GRP, L), jnp.float32),
            pltpu.VMEM_SHARED((NACC, L), jnp.float32),
            pltpu.SemaphoreType.DMA,
            pltpu.SemaphoreType.DMA,
        ],
    )
    return fn(table, sd2d, zeros_hbm)


def _sc_agg4(t0, t1, t2, t3, sd2d, zeros_hbm):
    fn = pl.kernel(
        _agg4_body,
        out_type=jax.ShapeDtypeStruct((4, NC, NACC, L), jnp.float32),
        mesh=_sc_mesh(),
        compiler_params=_SC_PARAMS,
        scratch_types=[
            pltpu.VMEM((4 * K, 128), jnp.int32),
            pltpu.VMEM((2 * GRP, L), jnp.float32),
            pltpu.VMEM_SHARED((NACC, L), jnp.float32),
            pltpu.SemaphoreType.DMA,
            pltpu.SemaphoreType.DMA,
        ],
    )
    return fn(t0, t1, t2, t3, sd2d, zeros_hbm)


PBLK = 256
GRID = PK // PBLK


def _pspec():
    return pl.BlockSpec((PBLK, 128), lambda i: (i, 0))


def _full(shape):
    return pl.BlockSpec(shape, lambda i: tuple(0 for _ in shape))


def _tc_a_body(dp, xpk, bd_wtop, xs_ref, dinv_ref, xtop_ref):
    dinv = lax.rsqrt(dp[0] + dp[1] + 1.0)
    dinv_ref[...] = dinv
    xs_ref[...] = xpk[...] * dinv
    xtop_ref[...] = jnp.dot(xpk[...], bd_wtop[...],
                            preferred_element_type=jnp.float32)


def _tc_a(degp_p, xpk, bd_wtop):
    return pl.pallas_call(
        _tc_a_body,
        grid=(GRID,),
        in_specs=[
            pl.BlockSpec((NC, PBLK, 128), lambda i: (0, i, 0)),
            _pspec(),
            _full((128, 8)),
        ],
        out_specs=[_pspec(), _pspec(), pl.BlockSpec((PBLK, 8), lambda i: (i, 0))],
        out_shape=[
            jax.ShapeDtypeStruct((PK, 128), jnp.float32),
            jax.ShapeDtypeStruct((PK, 128), jnp.float32),
            jax.ShapeDtypeStruct((PK, 8), jnp.float32),
        ],
    )(degp_p, xpk, bd_wtop)


def _tc_b_body(a1p, xs, dinv, bd_w1, bd_m, b1t, sel0, sel1, sel2, sel3,
               h0, h1, h2, h3):
    u = (a1p[0] + a1p[1] + xs[...]) * dinv[...]
    h = jnp.dot(u, bd_w1[...], preferred_element_type=jnp.float32) + b1t[...]
    h = jnp.maximum(h, 0.0)
    dinv64 = jnp.dot(dinv[...], bd_m[...], preferred_element_type=jnp.float32)
    hs = h * dinv64
    for ref, sel in ((h0, sel0), (h1, sel1), (h2, sel2), (h3, sel3)):
        ref[...] = jnp.dot(hs, sel[...], preferred_element_type=jnp.float32)


def _tc_b(a1p_p, xs, dinv, bd_w1, bd_m, b1t, sels):
    return pl.pallas_call(
        _tc_b_body,
        grid=(GRID,),
        in_specs=[
            pl.BlockSpec((NC, PBLK, 128), lambda i: (0, i, 0)),
            _pspec(), _pspec(),
            _full((128, 8 * HID)), _full((128, 8 * HID)), _full((1, 8 * HID)),
            _full((8 * HID, 128)), _full((8 * HID, 128)),
            _full((8 * HID, 128)), _full((8 * HID, 128)),
        ],
        out_specs=[_pspec()] * 4,
        out_shape=[jax.ShapeDtypeStruct((PK, 128), jnp.float32)] * 4,
    )(a1p_p, xs, dinv, bd_w1, bd_m, b1t, *sels)


def _tc_c_body(a2p, h0, h1, h2, h3, dinv, xtop, w0, w1, w2, w3, b2t, bd_wbot,
               bfc, out_ref):
    hs = (h0, h1, h2, h3)
    ws = (w0, w1, w2, w3)
    acc = b2t[...]
    for c in range(4):
        a2c = (a2p[c, 0] + a2p[c, 1] + hs[c][...]) * dinv[...]
        acc = acc + jnp.dot(a2c, ws[c][...],
                            preferred_element_type=jnp.float32)
    x2 = jnp.maximum(acc, 0.0)
    out_ref[...] = (xtop[...]
                    + jnp.dot(x2, bd_wbot[...],
                              preferred_element_type=jnp.float32)
                    + bfc[...])


def _tc_c(a2p_p, h1s_p, dinv, xtop, bd_w2, b2t, bd_wbot, bfc):
    return pl.pallas_call(
        _tc_c_body,
        grid=(GRID,),
        in_specs=[
            pl.BlockSpec((4, NC, PBLK, 128), lambda i: (0, 0, i, 0)),
            _pspec(), _pspec(), _pspec(), _pspec(),
            _pspec(),
            pl.BlockSpec((PBLK, 8), lambda i: (i, 0)),
            _full((128, 8 * 2 * HID)), _full((128, 8 * 2 * HID)),
            _full((128, 8 * 2 * HID)), _full((128, 8 * 2 * HID)),
            _full((1, 8 * 2 * HID)),
            _full((8 * 2 * HID, 8)),
            _full((1, 1)),
        ],
        out_specs=pl.BlockSpec((PBLK, 8), lambda i: (i, 0)),
        out_shape=jax.ShapeDtypeStruct((PK, 8), jnp.float32),
    )(a2p_p, *h1s_p, dinv, xtop, *bd_w2, b2t, bd_wbot, bfc)


def kernel(x, edge_index, W1, b1, W2, b2, Wfc, bfc):
    f32 = jnp.float32
    src = edge_index[0].astype(jnp.int32)
    dst = edge_index[1].astype(jnp.int32)
    npad = EPAD - E
    pad_i = jnp.arange(npad, dtype=jnp.int32)
    pad_src = (pad_i * 641) % N
    pad_dst = N + (pad_i % (NACC - N))
    dum_i = jnp.arange(32 * 2 * K * 128, dtype=jnp.int32)
    dum_src = (dum_i * 389) % N
    dum_dst = N + (dum_i % (NACC - N))
    s4 = jnp.concatenate([src, pad_src]).reshape(32, G32, K, 128)
    d4 = jnp.concatenate([dst, pad_dst]).reshape(32, G32, K, 128)
    dum4 = jnp.concatenate([dum_src.reshape(32, 2, K, 128),
                            dum_dst.reshape(32, 2, K, 128)], axis=2)
    sd2d = jnp.concatenate([
        jnp.concatenate([s4, d4], axis=2).reshape(32, G32 * 2 * K, 128),
        dum4.reshape(32, 4 * K, 128),
    ], axis=1).reshape(ROWS, 128)

    zeros_hbm = jnp.zeros((NACC, L), f32)
    ones_hbm = jnp.ones((128, L), f32)

    xpk = jnp.pad(x, ((0, NACC - N), (0, L - F_IN))).reshape(PK, 128)

    eye8 = jnp.eye(8, dtype=f32)
    wtop16 = jnp.pad(Wfc[:F_IN], ((0, L - F_IN), (0, 0)))
    bd_wtop = jnp.kron(eye8, wtop16)
    w1p = jnp.pad(W1, ((0, L - F_IN), (0, 0)))
    bd_w1 = jnp.kron(eye8, w1p)
    m16 = jnp.zeros((L, HID), f32).at[0, :].set(1.0)
    bd_m = jnp.kron(eye8, m16)
    b1t = jnp.tile(b1, 8).reshape(1, 8 * HID)
    sels = []
    for c in range(4):
        ec = jnp.zeros((HID, L), f32).at[c * L + jnp.arange(L),
                                         jnp.arange(L)].set(1.0)
        sels.append(jnp.kron(eye8, ec))
    bd_w2 = [jnp.kron(eye8, W2[c * L:(c + 1) * L]) for c in range(4)]
    b2t = jnp.tile(b2, 8).reshape(1, 8 * 2 * HID)
    bd_wbot = jnp.kron(eye8, Wfc[F_IN:])
    bfc2 = bfc.reshape(1, 1)

    degp = _sc_deg(sd2d, zeros_hbm, ones_hbm)
    degp_p = degp.reshape(NC, PK, 128)

    xs_p, dinv_p, xtop_p = _tc_a(degp_p, xpk, bd_wtop)

    a1p = _sc_agg(xs_p.reshape(NACC, L), sd2d, zeros_hbm)

    h1s_p = _tc_b(a1p.reshape(NC, PK, 128), xs_p, dinv_p, bd_w1, bd_m, b1t,
                  sels)

    a2p = _sc_agg4(h1s_p[0].reshape(NACC, L), h1s_p[1].reshape(NACC, L),
                   h1s_p[2].reshape(NACC, L), h1s_p[3].reshape(NACC, L),
                   sd2d, zeros_hbm)

    out = _tc_c(a2p.reshape(4, NC, PK, 128), h1s_p, dinv_p, xtop_p, bd_w2,
                b2t, bd_wbot, bfc2)
    return out.reshape(NACC)[:N]


# speedup vs baseline: 1.5605x; 1.0301x over previous
import jax
import jax.numpy as jnp
from jax import lax
from jax.experimental import pallas as pl
from jax.experimental.pallas import tpu as pltpu
from jax.experimental.pallas import tpu_sc as plsc

N = 100000
E = 1600000
F_IN = 10
HID = 64
NC, NS, L = 2, 16, 16

NACC = 102400
SLICE = NACC // NS
PK = NACC * L // 128

K = 6
GRP = K * 128
EPAD = 1622016
G32 = 66
RW = (G32 + 2) * 2 * K
ROWS = 32 * RW


def _zero_acc(acc, sid, zeros_hbm):
    pltpu.sync_copy(zeros_hbm.at[pl.ds(sid * SLICE, SLICE)],
                    acc.at[pl.ds(sid * SLICE, SLICE)])


def _copy_out(acc, out, cid, sid):
    pltpu.sync_copy(
        acc.at[pl.ds(sid * SLICE, SLICE)],
        out.at[cid, pl.ds(sid * SLICE, SLICE)],
    )


def _deg_body(sd_hbm, zeros_hbm, ones_hbm, out_hbm, didx2, ones, acc, isem):
    cid = lax.axis_index("c")
    sid = lax.axis_index("s")
    pltpu.sync_copy(ones_hbm, ones)
    _zero_acc(acc, sid, zeros_hbm)
    plsc.subcore_barrier()

    wid = cid * NS + sid
    base = wid * RW
    pltpu.async_copy(sd_hbm.at[pl.ds(base + K, K)], didx2.at[pl.ds(0, K)],
                     isem)

    def group(g, _):
        p = lax.rem(g, 2)
        o = p * K
        pltpu.make_async_copy(sd_hbm.at[pl.ds(0, K)],
                              didx2.at[pl.ds(o, K)], isem).wait()
        oo = K - o
        nrb = base + (g + 1) * 2 * K + K
        pltpu.async_copy(sd_hbm.at[pl.ds(nrb, K)], didx2.at[pl.ds(oo, K)],
                         isem)
        for j in range(K):
            pltpu.sync_copy(ones, acc.at[didx2.at[o + j]], add=True)
        return 0

    lax.fori_loop(0, G32, group, 0)
    pltpu.make_async_copy(sd_hbm.at[pl.ds(0, K)], didx2.at[pl.ds(0, K)],
                          isem).wait()
    plsc.subcore_barrier()
    _copy_out(acc, out_hbm, cid, sid)


def _agg_sweep(table_hbm, sd_hbm, zeros_hbm, acc, idx4, rows2, gsem, isem,
               wid):
    base = wid * RW
    pltpu.sync_copy(zeros_hbm.at[pl.ds(0, 2 * GRP)], rows2)
    pltpu.sync_copy(sd_hbm.at[pl.ds(base + G32 * 2 * K, 2 * K)],
                    idx4.at[pl.ds(2 * K, 2 * K)])
    pltpu.async_copy(sd_hbm.at[pl.ds(base, 2 * K)],
                     idx4.at[pl.ds(0, 2 * K)], isem)

    def step(g, _):
        p = lax.rem(g, 2)
        o = p * 2 * K
        ro = p * K
        pltpu.make_async_copy(sd_hbm.at[pl.ds(0, 2 * K)],
                              idx4.at[pl.ds(o, 2 * K)], isem).wait()
        handles = [
            pltpu.async_copy(table_hbm.at[idx4.at[o + j]],
                             rows2.at[pl.ds((ro + j) * 128, 128)], gsem)
            for j in range(K)
        ]
        oo = 2 * K - o
        roo = K - ro
        for j in range(K):
            pltpu.sync_copy(rows2.at[pl.ds((roo + j) * 128, 128)],
                            acc.at[idx4.at[oo + K + j]], add=True)
        pltpu.async_copy(sd_hbm.at[pl.ds(base + (g + 1) * 2 * K, 2 * K)],
                         idx4.at[pl.ds(oo, 2 * K)], isem)
        for h in handles:
            h.wait()
        return 0

    lax.fori_loop(0, G32 + 1, step, 0)
    pltpu.make_async_copy(sd_hbm.at[pl.ds(0, 2 * K)],
                          idx4.at[pl.ds(0, 2 * K)], isem).wait()


def _agg_body(table_hbm, sd_hbm, zeros_hbm, out_hbm, idx4, rows2, acc, gsem,
              isem):
    cid = lax.axis_index("c")
    sid = lax.axis_index("s")
    _zero_acc(acc, sid, zeros_hbm)
    plsc.subcore_barrier()

    wid = cid * NS + sid
    _agg_sweep(table_hbm, sd_hbm, zeros_hbm, acc, idx4, rows2, gsem, isem,
               wid)
    plsc.subcore_barrier()
    _copy_out(acc, out_hbm, cid, sid)


def _agg4_body(t0, t1, t2, t3, sd_hbm, zeros_hbm, out_hbm, idx4, rows2, acc,
               gsem, isem):
    cid = lax.axis_index("c")
    sid = lax.axis_index("s")
    wid = cid * NS + sid

    for c, table_hbm in enumerate((t0, t1, t2, t3)):
        _zero_acc(acc, sid, zeros_hbm)
        plsc.subcore_barrier()
        _agg_sweep(table_hbm, sd_hbm, zeros_hbm, acc, idx4, rows2, gsem,
                   isem, wid)
        plsc.subcore_barrier()
        pltpu.sync_copy(
            acc.at[pl.ds(sid * SLICE, SLICE)],
            out_hbm.at[c, cid, pl.ds(sid * SLICE, SLICE)],
        )
        plsc.subcore_barrier()


def _sc_mesh():
    return plsc.VectorSubcoreMesh(core_axis_name="c", subcore_axis_name="s")


_SC_PARAMS = pltpu.CompilerParams(use_tc_tiling_on_sc=False)


def _sc_deg(dst2d, zeros_hbm, ones_hbm):
    fn = pl.kernel(
        _deg_body,
        out_type=jax.ShapeDtypeStruct((NC, NACC, L), jnp.float32),
        mesh=_sc_mesh(),
        compiler_params=_SC_PARAMS,
        scratch_types=[
            pltpu.VMEM((2 * K, 128), jnp.int32),
            pltpu.VMEM((128, L), jnp.float32),
            pltpu.VMEM_SHARED((NACC, L), jnp.float32),
            pltpu.SemaphoreType.DMA,
        ],
    )
    return fn(dst2d, zeros_hbm, ones_hbm)


def _sc_agg(table, sd2d, zeros_hbm):
    fn = pl.kernel(
        _agg_body,
        out_type=jax.ShapeDtypeStruct((NC, NACC, L), jnp.float32),
        mesh=_sc_mesh(),
        compiler_params=_SC_PARAMS,
        scratch_types=[
            pltpu.VMEM((4 * K, 128), jnp.int32),
            pltpu.VMEM((2 * GRP, L), jnp.float32),
            pltpu.VMEM_SHARED((NACC, L), jnp.float32),
            pltpu.SemaphoreType.DMA,
            pltpu.SemaphoreType.DMA,
        ],
    )
    return fn(table, sd2d, zeros_hbm)


def _sc_agg4(t0, t1, t2, t3, sd2d, zeros_hbm):
    fn = pl.kernel(
        _agg4_body,
        out_type=jax.ShapeDtypeStruct((4, NC, NACC, L), jnp.float32),
        mesh=_sc_mesh(),
        compiler_params=_SC_PARAMS,
        scratch_types=[
            pltpu.VMEM((4 * K, 128), jnp.int32),
            pltpu.VMEM((2 * GRP, L), jnp.float32),
            pltpu.VMEM_SHARED((NACC, L), jnp.float32),
            pltpu.SemaphoreType.DMA,
            pltpu.SemaphoreType.DMA,
        ],
    )
    return fn(t0, t1, t2, t3, sd2d, zeros_hbm)


PBLK = 256
GRID = PK // PBLK


def _pspec():
    return pl.BlockSpec((PBLK, 128), lambda i: (i, 0))


def _full(shape):
    return pl.BlockSpec(shape, lambda i: tuple(0 for _ in shape))


def _tc_a_body(dp, xpk, bd_wtop, xs_ref, dinv_ref, xtop_ref):
    dinv = lax.rsqrt(dp[0] + dp[1] + 1.0)
    dinv_ref[...] = dinv
    xs_ref[...] = xpk[...] * dinv
    xtop_ref[...] = jnp.dot(xpk[...], bd_wtop[...],
                            preferred_element_type=jnp.float32)


def _tc_a(degp_p, xpk, bd_wtop):
    return pl.pallas_call(
        _tc_a_body,
        grid=(GRID,),
        in_specs=[
            pl.BlockSpec((NC, PBLK, 128), lambda i: (0, i, 0)),
            _pspec(),
            _full((128, 8)),
        ],
        out_specs=[_pspec(), _pspec(), pl.BlockSpec((PBLK, 8), lambda i: (i, 0))],
        out_shape=[
            jax.ShapeDtypeStruct((PK, 128), jnp.float32),
            jax.ShapeDtypeStruct((PK, 128), jnp.float32),
            jax.ShapeDtypeStruct((PK, 8), jnp.float32),
        ],
    )(degp_p, xpk, bd_wtop)


def _tc_b_body(a1p, xs, dinv, bd_w1, bd_m, b1t, sel0, sel1, sel2, sel3,
               h0, h1, h2, h3):
    u = (a1p[0] + a1p[1] + xs[...]) * dinv[...]
    h = jnp.dot(u, bd_w1[...], preferred_element_type=jnp.float32) + b1t[...]
    h = jnp.maximum(h, 0.0)
    dinv64 = jnp.dot(dinv[...], bd_m[...], preferred_element_type=jnp.float32)
    hs = h * dinv64
    for ref, sel in ((h0, sel0), (h1, sel1), (h2, sel2), (h3, sel3)):
        ref[...] = jnp.dot(hs, sel[...], preferred_element_type=jnp.float32)


def _tc_b(a1p_p, xs, dinv, bd_w1, bd_m, b1t, sels):
    return pl.pallas_call(
        _tc_b_body,
        grid=(GRID,),
        in_specs=[
            pl.BlockSpec((NC, PBLK, 128), lambda i: (0, i, 0)),
            _pspec(), _pspec(),
            _full((128, 8 * HID)), _full((128, 8 * HID)), _full((1, 8 * HID)),
            _full((8 * HID, 128)), _full((8 * HID, 128)),
            _full((8 * HID, 128)), _full((8 * HID, 128)),
        ],
        out_specs=[_pspec()] * 4,
        out_shape=[jax.ShapeDtypeStruct((PK, 128), jnp.float32)] * 4,
    )(a1p_p, xs, dinv, bd_w1, bd_m, b1t, *sels)


def _tc_c_body(a2p, h0, h1, h2, h3, dinv, xtop, w0, w1, w2, w3, b2t, bd_wbot,
               bfc, out_ref):
    hs = (h0, h1, h2, h3)
    ws = (w0, w1, w2, w3)
    acc = b2t[...]
    for c in range(4):
        a2c = (a2p[c, 0] + a2p[c, 1] + hs[c][...]) * dinv[...]
        acc = acc + jnp.dot(a2c, ws[c][...],
                            preferred_element_type=jnp.float32)
    x2 = jnp.maximum(acc, 0.0)
    out_ref[...] = (xtop[...]
                    + jnp.dot(x2, bd_wbot[...],
                              preferred_element_type=jnp.float32)
                    + bfc[...])


def _tc_c(a2p_p, h1s_p, dinv, xtop, bd_w2, b2t, bd_wbot, bfc):
    return pl.pallas_call(
        _tc_c_body,
        grid=(GRID,),
        in_specs=[
            pl.BlockSpec((4, NC, PBLK, 128), lambda i: (0, 0, i, 0)),
            _pspec(), _pspec(), _pspec(), _pspec(),
            _pspec(),
            pl.BlockSpec((PBLK, 8), lambda i: (i, 0)),
            _full((128, 8 * 2 * HID)), _full((128, 8 * 2 * HID)),
            _full((128, 8 * 2 * HID)), _full((128, 8 * 2 * HID)),
            _full((1, 8 * 2 * HID)),
            _full((8 * 2 * HID, 8)),
            _full((1, 1)),
        ],
        out_specs=pl.BlockSpec((PBLK, 8), lambda i: (i, 0)),
        out_shape=jax.ShapeDtypeStruct((PK, 8), jnp.float32),
    )(a2p_p, *h1s_p, dinv, xtop, *bd_w2, b2t, bd_wbot, bfc)


def kernel(x, edge_index, W1, b1, W2, b2, Wfc, bfc):
    f32 = jnp.float32
    src = edge_index[0].astype(jnp.int32)
    dst = edge_index[1].astype(jnp.int32)
    npad = EPAD - E
    pad_i = jnp.arange(npad, dtype=jnp.int32)
    pad_src = (pad_i * 641) % N
    pad_dst = N + (pad_i % (NACC - N))
    dum_i = jnp.arange(32 * 2 * K * 128, dtype=jnp.int32)
    dum_src = (dum_i * 389) % N
    dum_dst = N + (dum_i % (NACC - N))
    s4 = jnp.concatenate([src, pad_src]).reshape(32, G32, K, 128)
    d4 = jnp.concatenate([dst, pad_dst]).reshape(32, G32, K, 128)
    dum4 = jnp.concatenate([dum_src.reshape(32, 2, K, 128),
                            dum_dst.reshape(32, 2, K, 128)], axis=2)
    sd2d = jnp.concatenate([
        jnp.concatenate([s4, d4], axis=2).reshape(32, G32 * 2 * K, 128),
        dum4.reshape(32, 4 * K, 128),
    ], axis=1).reshape(ROWS, 128)

    zeros_hbm = jnp.zeros((NACC, L), f32)
    ones_hbm = jnp.ones((128, L), f32)

    xpk = jnp.pad(x, ((0, NACC - N), (0, L - F_IN))).reshape(PK, 128)

    eye8 = jnp.eye(8, dtype=f32)
    wtop16 = jnp.pad(Wfc[:F_IN], ((0, L - F_IN), (0, 0)))
    bd_wtop = jnp.kron(eye8, wtop16)
    w1p = jnp.pad(W1, ((0, L - F_IN), (0, 0)))
    bd_w1 = jnp.kron(eye8, w1p)
    m16 = jnp.zeros((L, HID), f32).at[0, :].set(1.0)
    bd_m = jnp.kron(eye8, m16)
    b1t = jnp.tile(b1, 8).reshape(1, 8 * HID)
    sels = []
    for c in range(4):
        ec = jnp.zeros((HID, L), f32).at[c * L + jnp.arange(L),
                                         jnp.arange(L)].set(1.0)
        sels.append(jnp.kron(eye8, ec))
    bd_w2 = [jnp.kron(eye8, W2[c * L:(c + 1) * L]) for c in range(4)]
    b2t = jnp.tile(b2, 8).reshape(1, 8 * 2 * HID)
    bd_wbot = jnp.kron(eye8, Wfc[F_IN:])
    bfc2 = bfc.reshape(1, 1)

    degp = _sc_deg(sd2d, zeros_hbm, ones_hbm)
    degp_p = degp.reshape(NC, PK, 128)

    xs_p, dinv_p, xtop_p = _tc_a(degp_p, xpk, bd_wtop)

    a1p = _sc_agg(xs_p.reshape(NACC, L), sd2d, zeros_hbm)

    h1s_p = _tc_b(a1p.reshape(NC, PK, 128), xs_p, dinv_p, bd_w1, bd_m, b1t,
                  sels)

    a2p = _sc_agg4(h1s_p[0].reshape(NACC, L), h1s_p[1].reshape(NACC, L),
                   h1s_p[2].reshape(NACC, L), h1s_p[3].reshape(NACC, L),
                   sd2d, zeros_hbm)

    out = _tc_c(a2p.reshape(4, NC, PK, 128), h1s_p, dinv_p, xtop_p, bd_w2,
                b2t, bd_wbot, bfc2)
    return out.reshape(NACC)[:N]
